# baseline (speedup 1.0000x reference)
import functools

import jax
import jax.numpy as jnp
from jax import lax
from jax.experimental import pallas as pl
from jax.experimental.pallas import tpu as pltpu
from jax.experimental.pallas import tpu_sc as plsc

NC = 2
NS = 16
NW = NC * NS
L = 16
R = 64
B = 16384
BPW = B // NW
CHUNK = 128
NCHUNK = BPW // CHUNK


def _body(idx_hbm, f0_hbm, f1_hbm, f2_hbm, out_hbm,
          idx_v, g0, g1, g2, out_v, sem):
    wid = lax.axis_index("s") * NC + lax.axis_index("c")
    base = wid * BPW

    for t in range(3):
        pltpu.sync_copy(idx_hbm.at[t, pl.ds(wid * NCHUNK, NCHUNK)],
                        idx_v.at[t])

    copies = []
    for t, (f_hbm, g) in enumerate(((f0_hbm, g0), (f1_hbm, g1), (f2_hbm, g2))):
        for c in range(NCHUNK):
            copies.append(pltpu.async_copy(
                f_hbm.at[idx_v.at[t, c]],
                g.at[pl.ds(c * CHUNK, CHUNK)],
                sem))
    for cp in copies:
        cp.wait()

    iota = lax.iota(jnp.int32, L)

    def group(gi, carry):
        row = gi * L + iota
        acc = jnp.zeros((L,), jnp.float32)
        for r in range(R):
            col = jnp.full((L,), r, jnp.int32)
            acc += (plsc.load_gather(g0, [row, col])
                    * plsc.load_gather(g1, [row, col])
                    * plsc.load_gather(g2, [row, col]))
        out_v[pl.ds(gi * L, L)] = acc
        return carry

    lax.fori_loop(0, BPW // L, group, 0)

    pltpu.sync_copy(out_v, out_hbm.at[pl.ds(base, BPW)])


@functools.partial(
    pl.kernel,
    out_type=jax.ShapeDtypeStruct((B,), jnp.float32),
    mesh=plsc.VectorSubcoreMesh(core_axis_name="c", subcore_axis_name="s",
                                num_cores=NC, num_subcores=NS),
    scratch_types=[
        pltpu.VMEM((3, NCHUNK, CHUNK), jnp.int32),
        pltpu.VMEM((BPW, R), jnp.float32),
        pltpu.VMEM((BPW, R), jnp.float32),
        pltpu.VMEM((BPW, R), jnp.float32),
        pltpu.VMEM((BPW,), jnp.float32),
        pltpu.SemaphoreType.DMA,
    ],
    compiler_params=pltpu.CompilerParams(needs_layout_passes=False,
                                         use_tc_tiling_on_sc=False),
)
def _cp_lookup(idx_hbm, f0_hbm, f1_hbm, f2_hbm, out_hbm,
               idx_v, g0, g1, g2, out_v, sem):
    _body(idx_hbm, f0_hbm, f1_hbm, f2_hbm, out_hbm,
          idx_v, g0, g1, g2, out_v, sem)


def kernel(input, F0, F1, F2):
    idx = jnp.transpose(input.astype(jnp.int32)).reshape(3, B // CHUNK, CHUNK)
    return _cp_lookup(idx, F0, F1, F2)


# speedup vs baseline: 1.0408x; 1.0408x over previous
import functools

import jax
import jax.numpy as jnp
from jax import lax
from jax.experimental import pallas as pl
from jax.experimental.pallas import tpu as pltpu
from jax.experimental.pallas import tpu_sc as plsc

NC = 2
NS = 16
NW = NC * NS
L = 16
R = 64
B = 16384
BPW = B // NW
CHUNK = 128
NCHUNK = BPW // CHUNK


def _body(idx_hbm, f0_hbm, f1_hbm, f2_hbm, out_hbm,
          idx_v, g0, g1, g2, out_v, sem):
    wid = lax.axis_index("s") * NC + lax.axis_index("c")
    base = wid * BPW

    for t in range(3):
        pltpu.sync_copy(idx_hbm.at[t, pl.ds(wid * NCHUNK, NCHUNK)],
                        idx_v.at[t])

    copies = []
    for c in range(NCHUNK):
        for t, (f_hbm, g) in enumerate(((f0_hbm, g0), (f1_hbm, g1),
                                        (f2_hbm, g2))):
            copies.append(pltpu.async_copy(
                f_hbm.at[idx_v.at[t, c]],
                g.at[pl.ds(c * CHUNK, CHUNK)],
                sem))

    iota = lax.iota(jnp.int32, L)
    cols = [jnp.full((L,), r, jnp.int32) for r in range(R)]

    def group(gi, carry):
        row = gi * L + iota
        accs = [jnp.zeros((L,), jnp.float32) for _ in range(4)]
        for r in range(R):
            accs[r % 4] += (plsc.load_gather(g0, [row, cols[r]])
                            * plsc.load_gather(g1, [row, cols[r]])
                            * plsc.load_gather(g2, [row, cols[r]]))
        out_v[pl.ds(gi * L, L)] = (accs[0] + accs[1]) + (accs[2] + accs[3])
        return carry

    gpc = CHUNK // L
    for c in range(NCHUNK):
        for t in range(3):
            copies[c * 3 + t].wait()
        lax.fori_loop(c * gpc, (c + 1) * gpc, group, 0, unroll=2)

    pltpu.sync_copy(out_v, out_hbm.at[pl.ds(base, BPW)])


@functools.partial(
    pl.kernel,
    out_type=jax.ShapeDtypeStruct((B,), jnp.float32),
    mesh=plsc.VectorSubcoreMesh(core_axis_name="c", subcore_axis_name="s",
                                num_cores=NC, num_subcores=NS),
    scratch_types=[
        pltpu.VMEM((3, NCHUNK, CHUNK), jnp.int32),
        pltpu.VMEM((BPW, R), jnp.float32),
        pltpu.VMEM((BPW, R), jnp.float32),
        pltpu.VMEM((BPW, R), jnp.float32),
        pltpu.VMEM((BPW,), jnp.float32),
        pltpu.SemaphoreType.DMA,
    ],
    compiler_params=pltpu.CompilerParams(needs_layout_passes=False,
                                         use_tc_tiling_on_sc=False),
)
def _cp_lookup(idx_hbm, f0_hbm, f1_hbm, f2_hbm, out_hbm,
               idx_v, g0, g1, g2, out_v, sem):
    _body(idx_hbm, f0_hbm, f1_hbm, f2_hbm, out_hbm,
          idx_v, g0, g1, g2, out_v, sem)


def kernel(input, F0, F1, F2):
    idx = jnp.transpose(input.astype(jnp.int32)).reshape(3, B // CHUNK, CHUNK)
    return _cp_lookup(idx, F0, F1, F2)


# speedup vs baseline: 1.0584x; 1.0169x over previous
import functools

import jax
import jax.numpy as jnp
from jax import lax
from jax.experimental import pallas as pl
from jax.experimental.pallas import tpu as pltpu
from jax.experimental.pallas import tpu_sc as plsc

NC = 2
NS = 16
NW = NC * NS
L = 16
R = 64
B = 16384
BPW = B // NW
CHUNK = 128
NCHUNK = BPW // CHUNK
GPC = CHUNK // L


def _body(idx_hbm, f0_hbm, f1_hbm, f2_hbm, out_hbm,
          idx_v, pidx_v, bufs, out_v, sem):
    wid = lax.axis_index("s") * NC + lax.axis_index("c")
    base = wid * BPW
    tables = (f0_hbm, f1_hbm, f2_hbm)

    for t in range(3):
        pltpu.sync_copy(idx_hbm.at[t, pl.ds(wid * NCHUNK, NCHUNK)],
                        idx_v.at[t])

    for t in range(3):
        for j in range(NCHUNK):
            for k in range(GPC):
                v = idx_v[t, j, pl.ds(k * L, L)]
                pidx_v[t, j, pl.ds(k * L, L)] = v >> 1

    def fire(c):
        return [pltpu.async_copy(tables[t].at[pidx_v.at[t, c]],
                                 bufs[t][c % 2], sem)
                for t in range(3)]

    iota = lax.iota(jnp.int32, L)
    inflight = fire(0)
    for c in range(NCHUNK):
        nxt = fire(c + 1) if c + 1 < NCHUNK else []
        for cp in inflight:
            cp.wait()
        g0, g1, g2 = (bufs[t][c % 2] for t in range(3))

        def group(k, carry):
            row = k * L + iota
            h = [(idx_v[t, c, pl.ds(k * L, L)] & 1) * R for t in range(3)]
            accs = [jnp.zeros((L,), jnp.float32) for _ in range(4)]
            for r in range(R):
                accs[r % 4] += (plsc.load_gather(g0, [row, h[0] + r])
                                * plsc.load_gather(g1, [row, h[1] + r])
                                * plsc.load_gather(g2, [row, h[2] + r]))
            out_v[pl.ds(c * CHUNK + k * L, L)] = ((accs[0] + accs[1])
                                                  + (accs[2] + accs[3]))
            return carry

        lax.fori_loop(0, GPC, group, 0)
        inflight = nxt

    pltpu.sync_copy(out_v, out_hbm.at[pl.ds(base, BPW)])


@functools.partial(
    pl.kernel,
    out_type=jax.ShapeDtypeStruct((B,), jnp.float32),
    mesh=plsc.VectorSubcoreMesh(core_axis_name="c", subcore_axis_name="s",
                                num_cores=NC, num_subcores=NS),
    scratch_types=[
        pltpu.VMEM((3, NCHUNK, CHUNK), jnp.int32),
        pltpu.VMEM((3, NCHUNK, CHUNK), jnp.int32),
    ] + [pltpu.VMEM((CHUNK, 2 * R), jnp.float32) for _ in range(6)] + [
        pltpu.VMEM((BPW,), jnp.float32),
        pltpu.SemaphoreType.DMA,
    ],
    compiler_params=pltpu.CompilerParams(needs_layout_passes=False,
                                         use_tc_tiling_on_sc=False),
)
def _cp_lookup(idx_hbm, f0_hbm, f1_hbm, f2_hbm, out_hbm,
               idx_v, pidx_v, b0a, b0b, b1a, b1b, b2a, b2b, out_v, sem):
    bufs = ((b0a, b0b), (b1a, b1b), (b2a, b2b))
    _body(idx_hbm, f0_hbm, f1_hbm, f2_hbm, out_hbm,
          idx_v, pidx_v, bufs, out_v, sem)


def kernel(input, F0, F1, F2):
    idx = jnp.transpose(input.astype(jnp.int32)).reshape(3, B // CHUNK, CHUNK)
    g0 = F0.reshape(50000, 2 * R)
    g1 = F1.reshape(50000, 2 * R)
    g2 = F2.reshape(50000, 2 * R)
    return _cp_lookup(idx, g0, g1, g2)


# speedup vs baseline: 1.0692x; 1.0102x over previous
import functools

import jax
import jax.numpy as jnp
from jax import lax
from jax.experimental import pallas as pl
from jax.experimental.pallas import tpu as pltpu
from jax.experimental.pallas import tpu_sc as plsc

NC = 2
NS = 16
NW = NC * NS
L = 16
R = 64
RP = 128
B = 16384
BPW = B // NW
CHUNK = 128
NCHUNK = BPW // CHUNK
GPC = CHUNK // L


def _body(idx_hbm, f0_hbm, f1_hbm, f2_hbm, out_hbm,
          idx_v, bufs, out_v, sem):
    wid = lax.axis_index("s") * NC + lax.axis_index("c")
    tables = (f0_hbm, f1_hbm, f2_hbm)

    for t in range(3):
        pltpu.sync_copy(idx_hbm.at[t, pl.ds(wid * NCHUNK, NCHUNK)],
                        idx_v.at[t])

    def fire(c):
        return [pltpu.async_copy(tables[t].at[idx_v.at[t, c]],
                                 bufs[t][c % 2], sem)
                for t in range(3)]

    iota = lax.iota(jnp.int32, L)
    cols = [jnp.full((L,), r, jnp.int32) for r in range(R)]
    inflight = fire(0)
    for c in range(NCHUNK):
        nxt = fire(c + 1) if c + 1 < NCHUNK else []
        for cp in inflight:
            cp.wait()
        g0, g1, g2 = (bufs[t][c % 2] for t in range(3))

        def group(k, carry):
            row = k * L + iota
            accs = [jnp.zeros((L,), jnp.float32) for _ in range(4)]
            for r in range(R):
                accs[r % 4] += (plsc.load_gather(g0, [row, cols[r]])
                                * plsc.load_gather(g1, [row, cols[r]])
                                * plsc.load_gather(g2, [row, cols[r]]))
            out_v[c, pl.ds(k * L, L)] = ((accs[0] + accs[1])
                                         + (accs[2] + accs[3]))
            return carry

        lax.fori_loop(0, GPC, group, 0)
        inflight = nxt

    pltpu.sync_copy(out_v, out_hbm.at[pl.ds(wid * NCHUNK, NCHUNK)])


@functools.partial(
    pl.kernel,
    out_type=jax.ShapeDtypeStruct((B // RP, RP), jnp.float32),
    mesh=plsc.VectorSubcoreMesh(core_axis_name="c", subcore_axis_name="s",
                                num_cores=NC, num_subcores=NS),
    scratch_types=[
        pltpu.VMEM((3, NCHUNK, CHUNK), jnp.int32),
    ] + [pltpu.VMEM((CHUNK, RP), jnp.float32) for _ in range(6)] + [
        pltpu.VMEM((NCHUNK, CHUNK), jnp.float32),
        pltpu.SemaphoreType.DMA,
    ],
    compiler_params=pltpu.CompilerParams(needs_layout_passes=False,
                                         use_tc_tiling_on_sc=True),
)
def _cp_lookup(idx_hbm, f0_hbm, f1_hbm, f2_hbm, out_hbm,
               idx_v, b0a, b0b, b1a, b1b, b2a, b2b, out_v, sem):
    bufs = ((b0a, b0b), (b1a, b1b), (b2a, b2b))
    _body(idx_hbm, f0_hbm, f1_hbm, f2_hbm, out_hbm,
          idx_v, bufs, out_v, sem)


def kernel(input, F0, F1, F2):
    idx = jnp.transpose(input.astype(jnp.int32)).reshape(3, B // CHUNK, CHUNK)
    pad = ((0, 0), (0, RP - R))
    out2d = _cp_lookup(idx, jnp.pad(F0, pad), jnp.pad(F1, pad),
                       jnp.pad(F2, pad))
    return out2d.reshape(B)
